# Initial kernel scaffold; baseline (speedup 1.0000x reference)
import jax
import jax.numpy as jnp
from jax.experimental import pallas as pl


def kernel(x, edge_index1, edge_index2, W_l1, b_l1, W_r1, W_l2, b_l2, W_r2):
    raise NotImplementedError("write your pallas kernel here")


# speedup vs baseline: 6.8377x; 6.8377x over previous
import functools

import jax
import jax.numpy as jnp
from jax import lax
from jax.experimental import pallas as pl
from jax.experimental.pallas import tpu as pltpu
from jax.experimental.pallas import tpu_sc as plsc

N0, N1, N2 = 10000, 5000, 2048
E1, E2 = 160000, 65536
D = 128

NC, NS = 2, 16
NT = NC * NS
K = 128
ZC = 64


def _make_sc_segsum(num_rows_pad, chunks_per_tile):
    rows_per_tile = num_rows_pad // NS
    assert rows_per_tile % ZC == 0

    @functools.partial(
        pl.kernel,
        mesh=plsc.VectorSubcoreMesh(core_axis_name="c", subcore_axis_name="s"),
        out_type=[
            jax.ShapeDtypeStruct((NC, num_rows_pad, D), jnp.float32),
            jax.ShapeDtypeStruct((NC, num_rows_pad, D), jnp.float32),
        ],
        scratch_types=[
            pltpu.VMEM((chunks_per_tile, K), jnp.int32),
            pltpu.VMEM((chunks_per_tile, K), jnp.int32),
            pltpu.VMEM((K, D), jnp.float32),
            pltpu.VMEM((K, D), jnp.float32),
            pltpu.SemaphoreType.DMA,
            pltpu.VMEM_SHARED((num_rows_pad, D), jnp.float32),
            pltpu.VMEM_SHARED((num_rows_pad, D), jnp.float32),
        ],
    )
    def segsum(table, srcs, dsts, z_d, ones_in, acc_out, cnt_out,
               idx_s, idx_d, rows, ones_v, sem, acc_sh, cnt_sh):
        c = lax.axis_index("c")
        s = lax.axis_index("s")
        tid = c * NS + s
        pltpu.sync_copy(srcs.at[pl.ds(tid * chunks_per_tile, chunks_per_tile)], idx_s)
        pltpu.sync_copy(dsts.at[pl.ds(tid * chunks_per_tile, chunks_per_tile)], idx_d)
        pltpu.sync_copy(ones_in, ones_v)

        zbase = s * rows_per_tile

        def zbody(i, carry):
            pltpu.sync_copy(z_d, acc_sh.at[pl.ds(zbase + i * ZC, ZC)])
            pltpu.sync_copy(z_d, cnt_sh.at[pl.ds(zbase + i * ZC, ZC)])
            return carry

        lax.fori_loop(0, rows_per_tile // ZC, zbody, 0)
        plsc.subcore_barrier()

        def ebody(j, carry):
            pltpu.async_copy(table.at[idx_s.at[j]], rows, sem).wait()
            pltpu.sync_copy(rows, acc_sh.at[idx_d.at[j]], add=True)
            pltpu.sync_copy(ones_v, cnt_sh.at[idx_d.at[j]], add=True)
            return carry

        lax.fori_loop(0, chunks_per_tile, ebody, 0)
        plsc.subcore_barrier()

        pltpu.sync_copy(acc_sh.at[pl.ds(zbase, rows_per_tile)],
                        acc_out.at[c, pl.ds(zbase, rows_per_tile)])
        pltpu.sync_copy(cnt_sh.at[pl.ds(zbase, rows_per_tile)],
                        cnt_out.at[c, pl.ds(zbase, rows_per_tile)])

    return segsum


def _tc_layer(acc, cnt, x_src, W_l, b_l, W_r, n_rows, blk, relu):
    grid = n_rows // blk

    def body(a_ref, c_ref, x_ref, wl_ref, b_ref, wr_ref, o_ref):
        a = a_ref[0] + a_ref[1]
        cv = c_ref[0, :, 0:1] + c_ref[1, :, 0:1]
        mean = a / jnp.maximum(cv, 1.0)
        h = (jnp.dot(mean, wl_ref[...], preferred_element_type=jnp.float32)
             + b_ref[...]
             + jnp.dot(x_ref[...], wr_ref[...], preferred_element_type=jnp.float32))
        if relu:
            h = jnp.maximum(h, 0.0)
        o_ref[...] = h

    return pl.pallas_call(
        body,
        grid=(grid,),
        in_specs=[
            pl.BlockSpec((2, blk, D), lambda i: (0, i, 0)),
            pl.BlockSpec((2, blk, D), lambda i: (0, i, 0)),
            pl.BlockSpec((blk, D), lambda i: (i, 0)),
            pl.BlockSpec((D, D), lambda i: (0, 0)),
            pl.BlockSpec((1, D), lambda i: (0, 0)),
            pl.BlockSpec((D, D), lambda i: (0, 0)),
        ],
        out_specs=pl.BlockSpec((blk, D), lambda i: (i, 0)),
        out_shape=jax.ShapeDtypeStruct((n_rows, D), jnp.float32),
    )(acc, cnt, x_src, W_l, b_l.reshape(1, D), W_r)


EP1 = 163840
P1 = 5120
CPT1 = EP1 // (NT * K)
P2 = 2048
CPT2 = E2 // (NT * K)

_segsum1 = _make_sc_segsum(P1, CPT1)
_segsum2 = _make_sc_segsum(P2, CPT2)


def kernel(x, edge_index1, edge_index2, W_l1, b_l1, W_r1, W_l2, b_l2, W_r2):
    z_d = jnp.zeros((ZC, D), jnp.float32)
    ones_in = jnp.ones((K, D), jnp.float32)

    pad1 = EP1 - E1
    ar = jnp.arange(pad1, dtype=jnp.int32)
    src1 = jnp.concatenate([edge_index1[0], ar % N1]).reshape(-1, K)
    dst1 = jnp.concatenate([edge_index1[1], N1 + ar % (P1 - N1)]).reshape(-1, K)
    src2 = edge_index2[0].reshape(-1, K)
    dst2 = edge_index2[1].reshape(-1, K)

    acc1, cnt1 = _segsum1(x, src1, dst1, z_d, ones_in)
    h = _tc_layer(acc1, cnt1, x, W_l1, b_l1, W_r1, N1, 1000, relu=True)
    acc2, cnt2 = _segsum2(h, src2, dst2, z_d, ones_in)
    h2 = _tc_layer(acc2, cnt2, h, W_l2, b_l2, W_r2, N2, 1024, relu=False)
    return (h2, h2, h)


# speedup vs baseline: 10.9828x; 1.6062x over previous
import functools

import jax
import jax.numpy as jnp
from jax import lax
from jax.experimental import pallas as pl
from jax.experimental.pallas import tpu as pltpu
from jax.experimental.pallas import tpu_sc as plsc

N0, N1, N2 = 10000, 5000, 2048
E1, E2 = 160000, 65536
D = 128

NC, NS = 2, 16
NT = NC * NS
K = 128
ZC = 64
LANES = 16


def _make_sc_segsum(num_rows_pad, chunks_per_tile):
    rows_per_tile = num_rows_pad // NS
    assert rows_per_tile % ZC == 0
    cnt_rows = num_rows_pad // 128
    assert cnt_rows <= rows_per_tile
    assert chunks_per_tile % 2 == 0
    hl = 8 if num_rows_pad > 4096 else LANES
    nhalf = LANES // hl

    @functools.partial(
        pl.kernel,
        mesh=plsc.VectorSubcoreMesh(core_axis_name="c", subcore_axis_name="s"),
        compiler_params=pltpu.CompilerParams(needs_layout_passes=False),
        out_type=[
            jax.ShapeDtypeStruct((NC, num_rows_pad, D), jnp.float32),
            jax.ShapeDtypeStruct((NT, cnt_rows, 128), jnp.float32),
        ],
        scratch_types=[
            pltpu.VMEM((chunks_per_tile, K), jnp.int32),
            pltpu.VMEM((chunks_per_tile, K), jnp.int32),
            pltpu.VMEM((K, D), jnp.float32),
            pltpu.VMEM((K, D), jnp.float32),
            pltpu.VMEM((hl, cnt_rows, 128), jnp.float32),
            pltpu.SemaphoreType.DMA,
            pltpu.SemaphoreType.DMA,
            pltpu.VMEM_SHARED((num_rows_pad, D), jnp.float32),
        ],
    )
    def segsum(table, srcs, dsts, z_d, acc_out, cnt_out,
               idx_s, idx_d, rows0, rows1, hist, sem0, sem1, acc_sh):
        c = lax.axis_index("c")
        s = lax.axis_index("s")
        tid = c * NS + s
        pltpu.sync_copy(srcs.at[pl.ds(tid * chunks_per_tile, chunks_per_tile)], idx_s)
        pltpu.sync_copy(dsts.at[pl.ds(tid * chunks_per_tile, chunks_per_tile)], idx_d)

        pltpu.async_copy(table.at[idx_s.at[0]], rows0, sem0)

        zbase = s * rows_per_tile

        def zbody(i, carry):
            pltpu.sync_copy(z_d, acc_sh.at[pl.ds(zbase + i * ZC, ZC)])
            return carry

        lax.fori_loop(0, rows_per_tile // ZC, zbody, 0)

        def hzero(l, carry):
            pltpu.sync_copy(acc_sh.at[pl.ds(zbase, cnt_rows)], hist.at[l])
            return carry

        lax.fori_loop(0, hl, hzero, 0)
        plsc.subcore_barrier()

        iota16 = lax.iota(jnp.int32, LANES)
        lanes = lax.rem(iota16, hl)
        one16 = jnp.ones((LANES,), jnp.float32)
        halves = [None] if nhalf == 1 else [
            (iota16 // hl) == t for t in range(nhalf)]

        def hist_chunk(j):
            def hbody(g, carry):
                idx16 = idx_d[j, pl.ds(g * LANES, LANES)]
                row = lax.shift_right_logical(idx16, 7)
                col = lax.bitwise_and(idx16, 127)
                for msk in halves:
                    v = plsc.load_gather(hist, [lanes, row, col], mask=msk)
                    plsc.store_scatter(hist, [lanes, row, col], v + one16,
                                       mask=msk)
                return carry
            lax.fori_loop(0, K // LANES, hbody, 0)

        def ebody(jj, carry):
            j0 = jj * 2
            pltpu.async_copy(table.at[idx_s.at[j0 + 1]], rows1, sem1)
            pltpu.make_async_copy(table.at[idx_s.at[j0]], rows0, sem0).wait()
            hist_chunk(j0)
            pltpu.sync_copy(rows0, acc_sh.at[idx_d.at[j0]], add=True)

            @pl.when(jj + 1 < chunks_per_tile // 2)
            def _():
                pltpu.async_copy(table.at[idx_s.at[j0 + 2]], rows0, sem0)

            pltpu.make_async_copy(table.at[idx_s.at[j0 + 1]], rows1, sem1).wait()
            hist_chunk(j0 + 1)
            pltpu.sync_copy(rows1, acc_sh.at[idx_d.at[j0 + 1]], add=True)
            return carry

        lax.fori_loop(0, chunks_per_tile // 2, ebody, 0)

        def rbody(g, carry):
            r = g // 8
            o = (g % 8) * LANES
            tot = hist[0, r, pl.ds(o, LANES)]
            for l in range(1, hl):
                tot = tot + hist[l, r, pl.ds(o, LANES)]
            hist[0, r, pl.ds(o, LANES)] = tot
            return carry

        lax.fori_loop(0, num_rows_pad // LANES, rbody, 0)
        plsc.subcore_barrier()

        pltpu.sync_copy(acc_sh.at[pl.ds(zbase, rows_per_tile)],
                        acc_out.at[c, pl.ds(zbase, rows_per_tile)])
        pltpu.sync_copy(hist.at[0], cnt_out.at[tid])

    return segsum


def _tc_layer(acc, cntb, x_src, W_l, b_l, W_r, n_rows, blk, relu):
    grid = n_rows // blk

    def body(a_ref, c_ref, x_ref, wl_ref, b_ref, wr_ref, o_ref):
        a = a_ref[0] + a_ref[1]
        mean = a / c_ref[...]
        h = (jnp.dot(mean, wl_ref[...], preferred_element_type=jnp.float32)
             + b_ref[...]
             + jnp.dot(x_ref[...], wr_ref[...], preferred_element_type=jnp.float32))
        if relu:
            h = jnp.maximum(h, 0.0)
        o_ref[...] = h

    return pl.pallas_call(
        body,
        grid=(grid,),
        in_specs=[
            pl.BlockSpec((2, blk, D), lambda i: (0, i, 0)),
            pl.BlockSpec((blk, D), lambda i: (i, 0)),
            pl.BlockSpec((blk, D), lambda i: (i, 0)),
            pl.BlockSpec((D, D), lambda i: (0, 0)),
            pl.BlockSpec((1, D), lambda i: (0, 0)),
            pl.BlockSpec((D, D), lambda i: (0, 0)),
        ],
        out_specs=pl.BlockSpec((blk, D), lambda i: (i, 0)),
        out_shape=jax.ShapeDtypeStruct((n_rows, D), jnp.float32),
    )(acc, cntb, x_src, W_l, b_l.reshape(1, D), W_r)


EP1 = 163840
P1 = 5120
CPT1 = EP1 // (NT * K)
P2 = 2048
CPT2 = E2 // (NT * K)

_segsum1 = _make_sc_segsum(P1, CPT1)
_segsum2 = _make_sc_segsum(P2, CPT2)


def _cnt_broadcast(cnt, n_rows):
    c = jnp.maximum(cnt.sum(0).reshape(-1)[:n_rows], 1.0)
    return jnp.broadcast_to(c[:, None], (n_rows, D))


def kernel(x, edge_index1, edge_index2, W_l1, b_l1, W_r1, W_l2, b_l2, W_r2):
    z_d = jnp.zeros((ZC, D), jnp.float32)

    pad1 = EP1 - E1
    ar = jnp.arange(pad1, dtype=jnp.int32)
    src1 = jnp.concatenate([edge_index1[0], ar % N1]).reshape(-1, K)
    dst1 = jnp.concatenate([edge_index1[1], N1 + ar % (P1 - N1)]).reshape(-1, K)
    src2 = edge_index2[0].reshape(-1, K)
    dst2 = edge_index2[1].reshape(-1, K)

    acc1, cnt1 = _segsum1(x, src1, dst1, z_d)
    h = _tc_layer(acc1, _cnt_broadcast(cnt1, N1), x, W_l1, b_l1, W_r1, N1, 1000, relu=True)
    acc2, cnt2 = _segsum2(h, src2, dst2, z_d)
    h2 = _tc_layer(acc2, _cnt_broadcast(cnt2, N2), h, W_l2, b_l2, W_r2, N2, 1024, relu=False)
    return (h2, h2, h)


# speedup vs baseline: 11.6633x; 1.0620x over previous
import functools

import jax
import jax.numpy as jnp
from jax import lax
from jax.experimental import pallas as pl
from jax.experimental.pallas import tpu as pltpu
from jax.experimental.pallas import tpu_sc as plsc

N0, N1, N2 = 10000, 5000, 2048
E1, E2 = 160000, 65536
D = 128

NC, NS = 2, 16
NT = NC * NS
K = 64
ZC = 64
LANES = 16


def _make_sc_segsum(num_rows_pad, chunks_per_tile):
    rows_per_tile = num_rows_pad // NS
    assert rows_per_tile % ZC == 0
    cnt_rows = num_rows_pad // 128
    assert cnt_rows <= rows_per_tile
    assert chunks_per_tile % 4 == 0
    hl = 4 if num_rows_pad > 4096 else LANES
    nhalf = LANES // hl

    @functools.partial(
        pl.kernel,
        mesh=plsc.VectorSubcoreMesh(core_axis_name="c", subcore_axis_name="s"),
        compiler_params=pltpu.CompilerParams(needs_layout_passes=False),
        out_type=[
            jax.ShapeDtypeStruct((NC, num_rows_pad, D), jnp.float32),
            jax.ShapeDtypeStruct((NT, cnt_rows, 128), jnp.float32),
        ],
        scratch_types=[
            pltpu.VMEM((chunks_per_tile, K), jnp.int32),
            pltpu.VMEM((chunks_per_tile, K), jnp.int32),
            pltpu.VMEM((K, D), jnp.float32),
            pltpu.VMEM((K, D), jnp.float32),
            pltpu.VMEM((K, D), jnp.float32),
            pltpu.VMEM((K, D), jnp.float32),
            pltpu.VMEM((hl, cnt_rows, 128), jnp.float32),
            pltpu.SemaphoreType.DMA,
            pltpu.SemaphoreType.DMA,
            pltpu.SemaphoreType.DMA,
            pltpu.SemaphoreType.DMA,
            pltpu.SemaphoreType.DMA,
            pltpu.SemaphoreType.DMA,
            pltpu.SemaphoreType.DMA,
            pltpu.SemaphoreType.DMA,
            pltpu.VMEM_SHARED((num_rows_pad, D), jnp.float32),
        ],
    )
    def segsum(table, srcs, dsts, z_d, acc_out, cnt_out,
               idx_s, idx_d, rows0, rows1, rows2, rows3, hist,
               sg0, sg1, sg2, sg3, ss0, ss1, ss2, ss3, acc_sh):
        rows = [rows0, rows1, rows2, rows3]
        sem_g = [sg0, sg1, sg2, sg3]
        sem_s = [ss0, ss1, ss2, ss3]
        c = lax.axis_index("c")
        s = lax.axis_index("s")
        tid = c * NS + s
        pltpu.sync_copy(srcs.at[pl.ds(tid * chunks_per_tile, chunks_per_tile)], idx_s)
        pltpu.sync_copy(dsts.at[pl.ds(tid * chunks_per_tile, chunks_per_tile)], idx_d)

        pltpu.async_copy(table.at[idx_s.at[0]], rows0, sg0)
        pltpu.async_copy(table.at[idx_s.at[1]], rows1, sg1)

        zbase = s * rows_per_tile

        def zbody(i, carry):
            pltpu.sync_copy(z_d, acc_sh.at[pl.ds(zbase + i * ZC, ZC)])
            return carry

        lax.fori_loop(0, rows_per_tile // ZC, zbody, 0)

        def hzero(l, carry):
            pltpu.sync_copy(acc_sh.at[pl.ds(zbase, cnt_rows)], hist.at[l])
            return carry

        lax.fori_loop(0, hl, hzero, 0)
        plsc.subcore_barrier()

        iota16 = lax.iota(jnp.int32, LANES)
        lanes = lax.rem(iota16, hl)
        one16 = jnp.ones((LANES,), jnp.float32)
        halves = [None] if nhalf == 1 else [
            (iota16 // hl) == t for t in range(nhalf)]

        def hist_chunk(j):
            def hbody(g, carry):
                idx16 = idx_d[j, pl.ds(g * LANES, LANES)]
                row = lax.shift_right_logical(idx16, 7)
                col = lax.bitwise_and(idx16, 127)
                for msk in halves:
                    v = plsc.load_gather(hist, [lanes, row, col], mask=msk)
                    plsc.store_scatter(hist, [lanes, row, col], v + one16,
                                       mask=msk)
                return carry
            lax.fori_loop(0, K // LANES, hbody, 0)

        def waitS(b, j):
            pltpu.make_async_copy(rows[b], acc_sh.at[idx_d.at[j]], sem_s[b]).wait()

        def ebody(jj, carry):
            j0 = jj * 4
            for u in range(4):
                j = j0 + u
                b2 = (u + 2) % 4

                @pl.when(j >= 2)
                def _():
                    waitS(b2, j - 2)

                @pl.when(j + 2 < chunks_per_tile)
                def _():
                    pltpu.async_copy(table.at[idx_s.at[j + 2]], rows[b2], sem_g[b2])

                pltpu.make_async_copy(table.at[idx_s.at[j]], rows[u], sem_g[u]).wait()
                pltpu.async_copy(rows[u], acc_sh.at[idx_d.at[j]], sem_s[u], add=True)
                hist_chunk(j)
            return carry

        lax.fori_loop(0, chunks_per_tile // 4, ebody, 0)
        waitS(2, chunks_per_tile - 2)
        waitS(3, chunks_per_tile - 1)

        def rbody(g, carry):
            r = g // 8
            o = (g % 8) * LANES
            tot = hist[0, r, pl.ds(o, LANES)]
            for l in range(1, hl):
                tot = tot + hist[l, r, pl.ds(o, LANES)]
            hist[0, r, pl.ds(o, LANES)] = tot
            return carry

        lax.fori_loop(0, num_rows_pad // LANES, rbody, 0)
        plsc.subcore_barrier()

        pltpu.sync_copy(acc_sh.at[pl.ds(zbase, rows_per_tile)],
                        acc_out.at[c, pl.ds(zbase, rows_per_tile)])
        pltpu.sync_copy(hist.at[0], cnt_out.at[tid])

    return segsum


def _tc_layer(acc, cntb, x_src, W_l, b_l, W_r, n_rows, blk, relu, row0=0):
    grid = n_rows // blk
    assert row0 % blk == 0
    r0 = row0 // blk

    def body(a_ref, c_ref, x_ref, wl_ref, b_ref, wr_ref, o_ref):
        a = a_ref[0] + a_ref[1]
        mean = a / c_ref[...]
        h = (jnp.dot(mean, wl_ref[...], preferred_element_type=jnp.float32)
             + b_ref[...]
             + jnp.dot(x_ref[...], wr_ref[...], preferred_element_type=jnp.float32))
        if relu:
            h = jnp.maximum(h, 0.0)
        o_ref[...] = h

    return pl.pallas_call(
        body,
        grid=(grid,),
        in_specs=[
            pl.BlockSpec((2, blk, D), lambda i: (0, i + r0, 0)),
            pl.BlockSpec((blk, D), lambda i: (i + r0, 0)),
            pl.BlockSpec((blk, D), lambda i: (i + r0, 0)),
            pl.BlockSpec((D, D), lambda i: (0, 0)),
            pl.BlockSpec((1, D), lambda i: (0, 0)),
            pl.BlockSpec((D, D), lambda i: (0, 0)),
        ],
        out_specs=pl.BlockSpec((blk, D), lambda i: (i, 0)),
        out_shape=jax.ShapeDtypeStruct((n_rows, D), jnp.float32),
    )(acc, cntb, x_src, W_l, b_l.reshape(1, D), W_r)


EP1 = 163840
P1 = 5120
CPT1 = EP1 // (NT * K)
P2 = 2048
CPT2 = E2 // (NT * K)

_segsum1 = _make_sc_segsum(P1, CPT1)
_segsum2 = _make_sc_segsum(P2, CPT2)


def _cnt_broadcast(cnt, n_rows):
    c = jnp.maximum(cnt.sum(0).reshape(-1)[:n_rows], 1.0)
    return jnp.broadcast_to(c[:, None], (n_rows, D))


def kernel(x, edge_index1, edge_index2, W_l1, b_l1, W_r1, W_l2, b_l2, W_r2):
    z_d = jnp.zeros((ZC, D), jnp.float32)

    pad1 = EP1 - E1
    ar = jnp.arange(pad1, dtype=jnp.int32)
    src1 = jnp.concatenate([edge_index1[0], ar % N1]).reshape(-1, K)
    dst1 = jnp.concatenate([edge_index1[1], N1 + ar % (P1 - N1)]).reshape(-1, K)
    src2 = edge_index2[0].reshape(-1, K)
    dst2 = edge_index2[1].reshape(-1, K)

    acc1, cnt1 = _segsum1(x, src1, dst1, z_d)
    cntb1 = _cnt_broadcast(cnt1, P1)
    h_head = _tc_layer(acc1, cntb1, x, W_l1, b_l1, W_r1, N2, 1024, relu=True)
    h_tail = _tc_layer(acc1, cntb1, x, W_l1, b_l1, W_r1, P1 - N2, 1024,
                       relu=True, row0=N2)
    acc2, cnt2 = _segsum2(h_head, src2, dst2, z_d)
    h2 = _tc_layer(acc2, _cnt_broadcast(cnt2, N2), h_head, W_l2, b_l2, W_r2,
                   N2, 1024, relu=False)
    h = jnp.concatenate([h_head, h_tail])[:N1]
    return (h2, h2, h)


# speedup vs baseline: 12.2814x; 1.0530x over previous
import functools

import jax
import jax.numpy as jnp
from jax import lax
from jax.experimental import pallas as pl
from jax.experimental.pallas import tpu as pltpu
from jax.experimental.pallas import tpu_sc as plsc

N0, N1, N2 = 10000, 5000, 2048
E1, E2 = 160000, 65536
D = 128

NC, NS = 2, 16
NT = NC * NS
K = 64
ZC = 64
LANES = 16


def _make_sc_segsum(num_rows_pad, chunks_per_tile):
    rows_per_tile = num_rows_pad // NS
    assert rows_per_tile % ZC == 0
    cnt_rows = num_rows_pad // 128
    assert cnt_rows <= rows_per_tile
    assert chunks_per_tile % 4 == 0
    hl = 4 if num_rows_pad > 4096 else LANES
    nhalf = LANES // hl

    @functools.partial(
        pl.kernel,
        mesh=plsc.VectorSubcoreMesh(core_axis_name="c", subcore_axis_name="s"),
        compiler_params=pltpu.CompilerParams(needs_layout_passes=False),
        out_type=[
            jax.ShapeDtypeStruct((NC, num_rows_pad, D), jnp.float32),
            jax.ShapeDtypeStruct((NT, cnt_rows, 128), jnp.float32),
        ],
        scratch_types=[
            pltpu.VMEM((chunks_per_tile, K), jnp.int32),
            pltpu.VMEM((chunks_per_tile, K), jnp.int32),
            pltpu.VMEM((K, D), jnp.float32),
            pltpu.VMEM((K, D), jnp.float32),
            pltpu.VMEM((K, D), jnp.float32),
            pltpu.VMEM((K, D), jnp.float32),
            pltpu.VMEM((hl * cnt_rows, 128), jnp.float32),
            pltpu.SemaphoreType.DMA,
            pltpu.SemaphoreType.DMA,
            pltpu.SemaphoreType.DMA,
            pltpu.SemaphoreType.DMA,
            pltpu.SemaphoreType.DMA,
            pltpu.SemaphoreType.DMA,
            pltpu.SemaphoreType.DMA,
            pltpu.SemaphoreType.DMA,
            pltpu.SemaphoreType.DMA,
            pltpu.VMEM_SHARED((num_rows_pad, D), jnp.float32),
        ],
    )
    def segsum(table, srcs, dsts, z_d, acc_out, cnt_out,
               idx_s, idx_d, rows0, rows1, rows2, rows3, hist,
               sg0, sg1, sg2, sg3, ss0, ss1, ss2, ss3, sem_p, acc_sh):
        rows = [rows0, rows1, rows2, rows3]
        sem_g = [sg0, sg1, sg2, sg3]
        sem_s = [ss0, ss1, ss2, ss3]
        c = lax.axis_index("c")
        s = lax.axis_index("s")
        tid = c * NS + s
        pltpu.sync_copy(srcs.at[pl.ds(tid * chunks_per_tile, chunks_per_tile)], idx_s)
        pltpu.sync_copy(dsts.at[pl.ds(tid * chunks_per_tile, chunks_per_tile)], idx_d)

        pltpu.async_copy(table.at[idx_s.at[0]], rows0, sg0)
        pltpu.async_copy(table.at[idx_s.at[1]], rows1, sg1)

        zbase = s * rows_per_tile
        nz = rows_per_tile // ZC
        for i in range(nz):
            pltpu.async_copy(z_d, acc_sh.at[pl.ds(zbase + i * ZC, ZC)], sem_p)
        hist_rows = hl * cnt_rows
        hz = []
        r = 0
        while r < hist_rows:
            n = min(ZC, hist_rows - r)
            hz.append((r, n))
            r += n
        for r, n in hz:
            pltpu.async_copy(z_d.at[pl.ds(0, n)], hist.at[pl.ds(r, n)], sem_p)
        for i in range(nz):
            pltpu.make_async_copy(z_d, acc_sh.at[pl.ds(zbase + i * ZC, ZC)], sem_p).wait()
        for r, n in hz:
            pltpu.make_async_copy(z_d.at[pl.ds(0, n)], hist.at[pl.ds(r, n)], sem_p).wait()
        plsc.subcore_barrier()

        iota16 = lax.iota(jnp.int32, LANES)
        lanes = lax.rem(iota16, hl)
        one16 = jnp.ones((LANES,), jnp.float32)
        halves = [None] if nhalf == 1 else [
            (iota16 // hl) == t for t in range(nhalf)]

        lrows = lanes * cnt_rows

        def hist_chunk(j):
            def hbody(g, carry):
                idx16 = idx_d[j, pl.ds(g * LANES, LANES)]
                row = lrows + lax.shift_right_logical(idx16, 7)
                col = lax.bitwise_and(idx16, 127)
                for msk in halves:
                    v = plsc.load_gather(hist, [row, col], mask=msk)
                    plsc.store_scatter(hist, [row, col], v + one16, mask=msk)
                return carry
            lax.fori_loop(0, K // LANES, hbody, 0)

        def waitS(b, j):
            pltpu.make_async_copy(rows[b], acc_sh.at[idx_d.at[j]], sem_s[b]).wait()

        def ebody(jj, carry):
            j0 = jj * 4
            for u in range(4):
                j = j0 + u
                b2 = (u + 2) % 4

                @pl.when(j >= 2)
                def _():
                    waitS(b2, j - 2)

                @pl.when(j + 2 < chunks_per_tile)
                def _():
                    pltpu.async_copy(table.at[idx_s.at[j + 2]], rows[b2], sem_g[b2])

                pltpu.make_async_copy(table.at[idx_s.at[j]], rows[u], sem_g[u]).wait()
                pltpu.async_copy(rows[u], acc_sh.at[idx_d.at[j]], sem_s[u], add=True)
                hist_chunk(j)
            return carry

        lax.fori_loop(0, chunks_per_tile // 4, ebody, 0)
        waitS(2, chunks_per_tile - 2)
        waitS(3, chunks_per_tile - 1)

        def rbody(g, carry):
            r = g // 8
            o = (g % 8) * LANES
            tot = hist[r, pl.ds(o, LANES)]
            for l in range(1, hl):
                tot = tot + hist[l * cnt_rows + r, pl.ds(o, LANES)]
            hist[r, pl.ds(o, LANES)] = tot
            return carry

        lax.fori_loop(0, num_rows_pad // LANES, rbody, 0)
        plsc.subcore_barrier()

        pltpu.async_copy(acc_sh.at[pl.ds(zbase, rows_per_tile)],
                         acc_out.at[c, pl.ds(zbase, rows_per_tile)], sem_p)
        pltpu.async_copy(hist.at[pl.ds(0, cnt_rows)], cnt_out.at[tid], sem_p)
        pltpu.make_async_copy(acc_sh.at[pl.ds(zbase, rows_per_tile)],
                              acc_out.at[c, pl.ds(zbase, rows_per_tile)], sem_p).wait()
        pltpu.make_async_copy(hist.at[pl.ds(0, cnt_rows)], cnt_out.at[tid], sem_p).wait()

    return segsum


def _tc_layer(acc, cntb, x_src, W_l, b_l, W_r, n_rows, blk, relu, row0=0):
    grid = n_rows // blk
    assert row0 % blk == 0
    r0 = row0 // blk

    def body(a_ref, c_ref, x_ref, wl_ref, b_ref, wr_ref, o_ref):
        a = a_ref[0] + a_ref[1]
        mean = a / c_ref[...]
        h = (jnp.dot(mean, wl_ref[...], preferred_element_type=jnp.float32)
             + b_ref[...]
             + jnp.dot(x_ref[...], wr_ref[...], preferred_element_type=jnp.float32))
        if relu:
            h = jnp.maximum(h, 0.0)
        o_ref[...] = h

    return pl.pallas_call(
        body,
        grid=(grid,),
        in_specs=[
            pl.BlockSpec((2, blk, D), lambda i: (0, i + r0, 0)),
            pl.BlockSpec((blk, D), lambda i: (i + r0, 0)),
            pl.BlockSpec((blk, D), lambda i: (i + r0, 0)),
            pl.BlockSpec((D, D), lambda i: (0, 0)),
            pl.BlockSpec((1, D), lambda i: (0, 0)),
            pl.BlockSpec((D, D), lambda i: (0, 0)),
        ],
        out_specs=pl.BlockSpec((blk, D), lambda i: (i, 0)),
        out_shape=jax.ShapeDtypeStruct((n_rows, D), jnp.float32),
    )(acc, cntb, x_src, W_l, b_l.reshape(1, D), W_r)


EP1 = 163840
P1 = 5120
CPT1 = EP1 // (NT * K)
P2 = 2048
CPT2 = E2 // (NT * K)

_segsum1 = _make_sc_segsum(P1, CPT1)
_segsum2 = _make_sc_segsum(P2, CPT2)


def _cnt_broadcast(cnt, n_rows):
    c = jnp.maximum(cnt.sum(0).reshape(-1)[:n_rows], 1.0)
    return jnp.broadcast_to(c[:, None], (n_rows, D))


def kernel(x, edge_index1, edge_index2, W_l1, b_l1, W_r1, W_l2, b_l2, W_r2):
    z_d = jnp.zeros((ZC, D), jnp.float32)

    pad1 = EP1 - E1
    ar = jnp.arange(pad1, dtype=jnp.int32)
    src1 = jnp.concatenate([edge_index1[0], ar % N1]).reshape(-1, K)
    dst1 = jnp.concatenate([edge_index1[1], N1 + ar % (P1 - N1)]).reshape(-1, K)
    src2 = edge_index2[0].reshape(-1, K)
    dst2 = edge_index2[1].reshape(-1, K)

    acc1, cnt1 = _segsum1(x, src1, dst1, z_d)
    cntb1 = _cnt_broadcast(cnt1, P1)
    h_head = _tc_layer(acc1, cntb1, x, W_l1, b_l1, W_r1, N2, 1024, relu=True)
    h_tail = _tc_layer(acc1, cntb1, x, W_l1, b_l1, W_r1, P1 - N2, 1024,
                       relu=True, row0=N2)
    acc2, cnt2 = _segsum2(h_head, src2, dst2, z_d)
    h2 = _tc_layer(acc2, _cnt_broadcast(cnt2, N2), h_head, W_l2, b_l2, W_r2,
                   N2, 1024, relu=False)
    h = jnp.concatenate([h_head, h_tail])[:N1]
    return (h2, h2, h)
